# baseline (device time: 35142 ns/iter reference)
import jax
import jax.numpy as jnp
from jax import lax
from jax.experimental import pallas as pl
from jax.experimental.pallas import tpu as pltpu

M = 1024
N = 1024
D = 4096
H = M // 2
WB = 4
NB = N // WB
KC = 8
CW = N // KC
SUB = KC // WB


def kernel(dy, W):
    def body(dy_ref, w_ref, out_ref, wvmem, dybuf, pbuf, ybuf,
             w_sems, dy_sem, out_sems,
             ysend_sems, yrecv_sems, xsend_sems, xrecv_sems):
        my_x = lax.axis_index("x")
        my_y = lax.axis_index("y")

        barrier_sem = pltpu.get_barrier_semaphore()
        pl.semaphore_signal(
            barrier_sem, inc=1,
            device_id=(my_x, 1 - my_y), device_id_type=pl.DeviceIdType.MESH)
        pl.semaphore_signal(
            barrier_sem, inc=1,
            device_id=(1 - my_x, my_y), device_id_type=pl.DeviceIdType.MESH)

        row0 = my_x * H

        dy_load = pltpu.make_async_copy(
            dy_ref.at[pl.ds(row0, H)], dybuf, dy_sem)
        dy_load.start()

        def w_load(b):
            return pltpu.make_async_copy(
                w_ref.at[pl.ds(b * NB, NB)], wvmem.at[pl.ds(b * NB, NB)],
                w_sems.at[b])

        for b in range(WB):
            w_load(b).start()

        def y_copy(k):
            return pltpu.make_async_remote_copy(
                src_ref=pbuf.at[k],
                dst_ref=ybuf.at[k],
                send_sem=ysend_sems.at[k],
                recv_sem=yrecv_sems.at[k],
                device_id=(my_x, 1 - my_y),
                device_id_type=pl.DeviceIdType.MESH,
            )

        def x_copy(k):
            return pltpu.make_async_remote_copy(
                src_ref=pbuf.at[k],
                dst_ref=out_ref.at[pl.ds(row0, H), pl.ds(k * CW, CW)],
                send_sem=xsend_sems.at[k],
                recv_sem=xrecv_sems.at[k],
                device_id=(1 - my_x, my_y),
                device_id_type=pl.DeviceIdType.MESH,
            )

        def out_copy(k):
            return pltpu.make_async_copy(
                pbuf.at[k],
                out_ref.at[pl.ds(row0, H), pl.ds(k * CW, CW)],
                out_sems.at[k],
            )

        dy_load.wait()
        for b in range(WB):
            w_load(b).wait()
            p = lax.dot_general(
                dybuf[...], wvmem[pl.ds(b * NB, NB), :],
                dimension_numbers=(((1,), (1,)), ((), ())),
                preferred_element_type=jnp.float32,
            )
            pb16 = p.astype(jnp.bfloat16)
            for s in range(SUB):
                k = b * SUB + s
                pbuf[k, :, :] = pb16[:, s * CW:(s + 1) * CW]
            if b == 0:
                pl.semaphore_wait(barrier_sem, 2)
            for s in range(SUB):
                y_copy(b * SUB + s).start()

        for k in range(KC):
            yc = y_copy(k)
            yc.wait_send()
            yc.wait_recv()
            pbuf[k, :, :] = pbuf[k, :, :] + ybuf[k, :, :]
            out_copy(k).start()
            x_copy(k).start()

        for k in range(KC):
            x_copy(k).wait()
            out_copy(k).wait()

    return pl.pallas_call(
        body,
        out_shape=jax.ShapeDtypeStruct((M, N), jnp.bfloat16),
        in_specs=[
            pl.BlockSpec(memory_space=pl.ANY),
            pl.BlockSpec(memory_space=pl.ANY),
        ],
        out_specs=pl.BlockSpec(memory_space=pl.ANY),
        scratch_shapes=[
            pltpu.VMEM((N, D), jnp.float32),
            pltpu.VMEM((H, D), jnp.float32),
            pltpu.VMEM((KC, H, CW), jnp.bfloat16),
            pltpu.VMEM((KC, H, CW), jnp.bfloat16),
            pltpu.SemaphoreType.DMA((WB,)),
            pltpu.SemaphoreType.DMA,
            pltpu.SemaphoreType.DMA((KC,)),
            pltpu.SemaphoreType.DMA((KC,)),
            pltpu.SemaphoreType.DMA((KC,)),
            pltpu.SemaphoreType.DMA((KC,)),
            pltpu.SemaphoreType.DMA((KC,)),
        ],
        compiler_params=pltpu.CompilerParams(collective_id=0),
    )(dy, W)
